# baseline (speedup 1.0000x reference)
import functools

import jax
import jax.numpy as jnp
from jax import lax
from jax.experimental import pallas as pl
from jax.experimental.pallas import tpu as pltpu
from jax.experimental.pallas import tpu_sc as plsc

BATCH = 16384
DIM = 64
VOCAB = 1000000

NUM_CORES = 2
NUM_SUBCORES = 16
LANES = 16
NW = NUM_CORES * NUM_SUBCORES
B_PER_W = BATCH // NW
N_CHUNK = 4
CHUNK = B_PER_W // N_CHUNK
G_PER_CHUNK = CHUNK // LANES

_mesh = plsc.VectorSubcoreMesh(core_axis_name="c", subcore_axis_name="s")


@functools.partial(
    pl.kernel,
    mesh=_mesh,
    out_type=jax.ShapeDtypeStruct((BATCH,), jnp.float32),
    compiler_params=pltpu.CompilerParams(
        needs_layout_passes=False, use_tc_tiling_on_sc=False),
    scratch_types=[
        pltpu.VMEM((N_CHUNK, CHUNK), jnp.int32),
        pltpu.VMEM((2, CHUNK, DIM), jnp.float32),
        pltpu.VMEM((DIM,), jnp.float32),
        pltpu.VMEM((B_PER_W,), jnp.float32),
        pltpu.SemaphoreType.DMA,
        pltpu.SemaphoreType.DMA,
    ],
)
def _lookup_dot(idx_hbm, table_hbm, w_hbm, out_hbm,
                idx_v, rows_v, w_v, y_v, sem0, sem1):
    wid = lax.axis_index("s") * NUM_CORES + lax.axis_index("c")
    base = wid * B_PER_W

    pltpu.sync_copy(w_hbm, w_v)
    for c in range(N_CHUNK):
        pltpu.sync_copy(idx_hbm.at[pl.ds(base + c * CHUNK, CHUNK)], idx_v.at[c])

    sems = [sem0, sem1]
    copies = [None, None]
    copies[0] = pltpu.async_copy(table_hbm.at[idx_v.at[0]], rows_v.at[0], sems[0])

    iota = lax.iota(jnp.int32, LANES)
    wchunks = [w_v[pl.ds(16 * j, 16)] for j in range(4)]

    for c in range(N_CHUNK):
        if c + 1 < N_CHUNK:
            nb = (c + 1) % 2
            copies[nb] = pltpu.async_copy(
                table_hbm.at[idx_v.at[c + 1]], rows_v.at[nb], sems[nb])
        copies[c % 2].wait()
        rows = rows_v.at[c % 2]

        def group_body(g, _, rows=rows, c=c):
            ys = jnp.zeros((LANES,), jnp.float32)
            for rl in range(LANES):
                r = g * LANES + rl
                p = rows[r, pl.ds(0, 16)] * wchunks[0]
                for j in range(1, 4):
                    p = p + rows[r, pl.ds(16 * j, 16)] * wchunks[j]
                ys = jnp.where(iota == rl, jnp.sum(p), ys)
            y_v[pl.ds(c * CHUNK + g * LANES, LANES)] = ys
            return 0

        lax.fori_loop(0, G_PER_CHUNK, group_body, 0)

    pltpu.sync_copy(y_v, out_hbm.at[pl.ds(base, B_PER_W)])


@jax.jit
def kernel(indices, table, w):
    idx = indices.astype(jnp.int32)
    y = _lookup_dot(idx, table, jnp.reshape(w, (DIM,)))
    return jnp.reshape(y, (BATCH, 1))


# speedup vs baseline: 5.4111x; 5.4111x over previous
import functools

import jax
import jax.numpy as jnp
from jax import lax
from jax.experimental import pallas as pl
from jax.experimental.pallas import tpu as pltpu
from jax.experimental.pallas import tpu_sc as plsc

BATCH = 16384
DIM = 64
VOCAB = 1000000

BLK = 16384
GRID = -(-VOCAB // BLK)

NUM_CORES = 2
NUM_SUBCORES = 16
NW = NUM_CORES * NUM_SUBCORES
B_PER_W = BATCH // NW
N_CHUNK = B_PER_W // 128


def _mv_body(tab_ref, w_ref, tv_ref):
    tv_ref[...] = jnp.sum(tab_ref[...] * w_ref[...], axis=0)


_matvec = pl.pallas_call(
    _mv_body,
    grid=(GRID,),
    in_specs=[
        pl.BlockSpec((DIM, BLK), lambda i: (0, i)),
        pl.BlockSpec((DIM, 1), lambda i: (0, 0)),
    ],
    out_specs=pl.BlockSpec((BLK,), lambda i: (i,)),
    out_shape=jax.ShapeDtypeStruct((VOCAB,), jnp.float32),
)

_mesh = plsc.VectorSubcoreMesh(core_axis_name="c", subcore_axis_name="s")


@functools.partial(
    pl.kernel,
    mesh=_mesh,
    out_type=jax.ShapeDtypeStruct((BATCH,), jnp.float32),
    compiler_params=pltpu.CompilerParams(
        needs_layout_passes=False, use_tc_tiling_on_sc=False),
    scratch_types=[
        pltpu.VMEM((N_CHUNK, 128), jnp.int32),
        pltpu.VMEM((N_CHUNK, 128), jnp.float32),
        pltpu.SemaphoreType.DMA,
    ],
)
def _lookup(idx_hbm, tv_hbm, out_hbm, idx_v, g_v, sem):
    wid = lax.axis_index("s") * NUM_CORES + lax.axis_index("c")
    base = wid * B_PER_W
    for c in range(N_CHUNK):
        pltpu.sync_copy(idx_hbm.at[pl.ds(base + c * 128, 128)], idx_v.at[c])
    copies = [
        pltpu.async_copy(tv_hbm.at[idx_v.at[c]], g_v.at[c], sem)
        for c in range(N_CHUNK)
    ]
    for c in range(N_CHUNK):
        copies[c].wait()
        pltpu.sync_copy(g_v.at[c], out_hbm.at[pl.ds(base + c * 128, 128)])


@jax.jit
def kernel(indices, table, w):
    idx = indices.astype(jnp.int32)
    tv = _matvec(jnp.transpose(table), w)
    y = _lookup(idx, tv)
    return jnp.reshape(y, (BATCH, 1))


# speedup vs baseline: 6.0025x; 1.1093x over previous
import functools

import jax
import jax.numpy as jnp
from jax import lax
from jax.experimental import pallas as pl
from jax.experimental.pallas import tpu as pltpu
from jax.experimental.pallas import tpu_sc as plsc

BATCH = 16384
DIM = 64
VOCAB = 1000000

BLK = 65536
GRID = -(-VOCAB // BLK)

NUM_CORES = 2
NUM_SUBCORES = 16
NW = NUM_CORES * NUM_SUBCORES
B_PER_W = BATCH // NW
N_CHUNK = B_PER_W // 128


def _mv_body(tab_ref, w_ref, tv_ref):
    tv_ref[...] = jnp.sum(tab_ref[...] * w_ref[...], axis=0)


_matvec = pl.pallas_call(
    _mv_body,
    grid=(GRID,),
    in_specs=[
        pl.BlockSpec((DIM, BLK), lambda i: (0, i)),
        pl.BlockSpec((DIM, 1), lambda i: (0, 0)),
    ],
    out_specs=pl.BlockSpec((BLK,), lambda i: (i,)),
    out_shape=jax.ShapeDtypeStruct((VOCAB,), jnp.float32),
)

_mesh = plsc.VectorSubcoreMesh(core_axis_name="c", subcore_axis_name="s")


@functools.partial(
    pl.kernel,
    mesh=_mesh,
    out_type=jax.ShapeDtypeStruct((BATCH,), jnp.float32),
    compiler_params=pltpu.CompilerParams(
        needs_layout_passes=False, use_tc_tiling_on_sc=False),
    scratch_types=[
        pltpu.VMEM((N_CHUNK, 128), jnp.int32),
        pltpu.VMEM((N_CHUNK, 128), jnp.float32),
        pltpu.SemaphoreType.DMA,
    ],
)
def _lookup(idx_hbm, tv_hbm, out_hbm, idx_v, g_v, sem):
    wid = lax.axis_index("s") * NUM_CORES + lax.axis_index("c")
    base = wid * B_PER_W
    for c in range(N_CHUNK):
        pltpu.sync_copy(idx_hbm.at[pl.ds(base + c * 128, 128)], idx_v.at[c])
    copies = [
        pltpu.async_copy(tv_hbm.at[idx_v.at[c]], g_v.at[c], sem)
        for c in range(N_CHUNK)
    ]
    for c in range(N_CHUNK):
        copies[c].wait()
        pltpu.sync_copy(g_v.at[c], out_hbm.at[pl.ds(base + c * 128, 128)])


@jax.jit
def kernel(indices, table, w):
    idx = indices.astype(jnp.int32)
    tv = _matvec(jnp.transpose(table), w)
    y = _lookup(idx, tv)
    return jnp.reshape(y, (BATCH, 1))
